# baseline (speedup 1.0000x reference)
import functools

import jax
import jax.numpy as jnp
from jax import lax
from jax.experimental import pallas as pl
from jax.experimental.pallas import tpu as pltpu

N = 10000
D = 128
BM = 400
NSTEPS = N // BM


def _body(x_ref, w_ref, b_ref, a_ref, adj_ref, o_ref, abuf, h_ref, sems):
    def block_copy(step, slot):
        return pltpu.make_async_copy(
            adj_ref.at[pl.ds(step * BM, BM), :], abuf.at[slot], sems.at[slot]
        )

    block_copy(0, 0).start()
    block_copy(1, 1).start()

    h = lax.dot_general(
        x_ref[...], w_ref[...], (((1,), (1,)), ((), ())),
        preferred_element_type=jnp.float32,
    ) + b_ref[...]
    h_ref[...] = h.astype(jnp.bfloat16)
    a = a_ref[0, 0]

    for i in range(NSTEPS):
        slot = i % 2
        block_copy(i, slot).wait()
        acc = jnp.dot(
            abuf[slot].astype(jnp.bfloat16), h_ref[...],
            preferred_element_type=jnp.float32,
        )
        o_ref[pl.ds(i * BM, BM), :] = jnp.where(acc >= 0, acc, a * acc)
        if i + 2 < NSTEPS:
            block_copy(i + 2, slot).start()


@functools.partial(jax.jit, static_argnames=())
def kernel(x, adj, W, b, a):
    x2 = x.reshape(N, D)
    b2 = b.reshape(1, D)
    a2 = a.reshape(1, 1)
    out = pl.pallas_call(
        _body,
        in_specs=[
            pl.BlockSpec((N, D), lambda: (0, 0)),
            pl.BlockSpec((D, D), lambda: (0, 0)),
            pl.BlockSpec((1, D), lambda: (0, 0)),
            pl.BlockSpec((1, 1), lambda: (0, 0)),
            pl.BlockSpec(memory_space=pl.ANY),
        ],
        out_specs=pl.BlockSpec((N, D), lambda: (0, 0)),
        out_shape=jax.ShapeDtypeStruct((N, D), jnp.float32),
        scratch_shapes=[
            pltpu.VMEM((2, BM, N), jnp.float32),
            pltpu.VMEM((N, D), jnp.bfloat16),
            pltpu.SemaphoreType.DMA((2,)),
        ],
        compiler_params=pltpu.CompilerParams(
            vmem_limit_bytes=64 * 1024 * 1024,
        ),
    )(x2, W, b2, a2, adj)
    return out.reshape(1, N, D)


# speedup vs baseline: 1.0596x; 1.0596x over previous
import functools

import jax
import jax.numpy as jnp
from jax import lax
from jax.experimental import pallas as pl
from jax.experimental.pallas import tpu as pltpu

N = 10000
D = 128
BM = 400


def _body(x_ref, w_ref, b_ref, a_ref, adj_ref, o_ref, h_ref):
    @pl.when(pl.program_id(0) == 0)
    def _():
        h = lax.dot_general(
            x_ref[...], w_ref[...], (((1,), (1,)), ((), ())),
            preferred_element_type=jnp.float32,
        ) + b_ref[...]
        h_ref[...] = h.astype(jnp.bfloat16)

    acc = jnp.dot(
        adj_ref[...].astype(jnp.bfloat16), h_ref[...],
        preferred_element_type=jnp.float32,
    )
    a = a_ref[0, 0]
    o_ref[...] = jnp.where(acc >= 0, acc, a * acc)


@functools.partial(jax.jit, static_argnames=())
def kernel(x, adj, W, b, a):
    x2 = x.reshape(N, D)
    b2 = b.reshape(1, D)
    a2 = a.reshape(1, 1)
    grid = (N // BM,)
    out = pl.pallas_call(
        _body,
        grid=grid,
        in_specs=[
            pl.BlockSpec((N, D), lambda i: (0, 0)),
            pl.BlockSpec((D, D), lambda i: (0, 0)),
            pl.BlockSpec((1, D), lambda i: (0, 0)),
            pl.BlockSpec((1, 1), lambda i: (0, 0)),
            pl.BlockSpec((BM, N), lambda i: (i, 0)),
        ],
        out_specs=pl.BlockSpec((BM, D), lambda i: (i, 0)),
        out_shape=jax.ShapeDtypeStruct((N, D), jnp.float32),
        scratch_shapes=[pltpu.VMEM((N, D), jnp.bfloat16)],
        compiler_params=pltpu.CompilerParams(
            dimension_semantics=("arbitrary",),
        ),
    )(x2, W, b2, a2, adj)
    return out.reshape(1, N, D)


# speedup vs baseline: 1.0695x; 1.0094x over previous
import functools

import jax
import jax.numpy as jnp
from jax import lax
from jax.experimental import pallas as pl
from jax.experimental.pallas import tpu as pltpu

N = 10000
D = 128
BM = 400


def _body(x_ref, w_ref, b_ref, a_ref, adj_ref, o_ref, h_ref):
    @pl.when(pl.program_id(0) == 0)
    def _():
        h = lax.dot_general(
            x_ref[...], w_ref[...], (((1,), (1,)), ((), ())),
            preferred_element_type=jnp.float32,
        ) + b_ref[...]
        h_ref[...] = h.astype(jnp.bfloat16)

    acc = jnp.dot(
        adj_ref[...].astype(jnp.bfloat16), h_ref[...],
        preferred_element_type=jnp.float32,
    )
    a = a_ref[0, 0]
    o_ref[...] = jnp.where(acc >= 0, acc, a * acc)


@functools.partial(jax.jit, static_argnames=())
def kernel(x, adj, W, b, a):
    x2 = x.reshape(N, D)
    b2 = b.reshape(1, D)
    a2 = a.reshape(1, 1)
    grid = (N // BM,)
    out = pl.pallas_call(
        _body,
        grid=grid,
        in_specs=[
            pl.BlockSpec((N, D), lambda i: (0, 0)),
            pl.BlockSpec((D, D), lambda i: (0, 0)),
            pl.BlockSpec((1, D), lambda i: (0, 0)),
            pl.BlockSpec((1, 1), lambda i: (0, 0)),
            pl.BlockSpec((BM, N), lambda i: (i, 0)),
        ],
        out_specs=pl.BlockSpec((BM, D), lambda i: (i, 0)),
        out_shape=jax.ShapeDtypeStruct((N, D), jnp.float32),
        scratch_shapes=[pltpu.VMEM((N, D), jnp.bfloat16)],
        compiler_params=pltpu.CompilerParams(
            dimension_semantics=("parallel",),
        ),
    )(x2, W, b2, a2, adj)
    return out.reshape(1, N, D)
